# baseline (speedup 1.0000x reference)
import functools

import jax
import jax.numpy as jnp
from jax import lax
from jax.experimental import pallas as pl
from jax.experimental.pallas import tpu as pltpu
from jax.experimental.pallas import tpu_sc as plsc

B = 1024
NUM_CLASS = 1000
K = 16
C = 512
PRE = 6
SUF = 55
T = PRE + K + SUF
NCONST = PRE + SUF
MASK_PROB = 0.5

NC, NS, L = 2, 16, 16
NW = NC * NS
BPW = B // NW
CL = C // L
REP = 16
HB = 8
NJ = (BPW // HB) * 2

_mesh = plsc.VectorSubcoreMesh(
    core_axis_name="c", subcore_axis_name="s", num_cores=NC, num_subcores=NS
)


@functools.partial(
    pl.kernel,
    out_type=jax.ShapeDtypeStruct((T, B, C), jnp.float32),
    mesh=_mesh,
    scratch_types=[
        pltpu.VMEM((NJ * HB,), jnp.int32),
        pltpu.VMEM((BPW,), jnp.float32),
        pltpu.VMEM((1, C), jnp.float32),
        pltpu.VMEM((HB, HB, C), jnp.float32),
        pltpu.VMEM((HB, HB, C), jnp.float32),
        pltpu.VMEM((1, HB, C), jnp.float32),
        pltpu.VMEM((1, HB, C), jnp.float32),
        pltpu.VMEM((1, PRE + 2 + SUF, C), jnp.float32),
        pltpu.VMEM((1, REP, C), jnp.float32),
        pltpu.VMEM((1, REP, C), jnp.float32),
        pltpu.SemaphoreType.DMA,
        pltpu.SemaphoreType.DMA,
        pltpu.SemaphoreType.DMA,
    ],
)
def _sc_prompts(gidx_hbm, ctx_hbm, pre_hbm, suf_hbm, dir_hbm, mask_hbm,
                out_hbm, gix, msk_v, d_v, gb0, gb1, ro0, ro1,
                cbuf, rp0, rp1, gsem, osem, csem):
    wid = lax.axis_index("s") * NC + lax.axis_index("c")
    base = wid * BPW

    pltpu.sync_copy(
        gidx_hbm.at[pl.ds(pl.multiple_of(wid * (NJ * HB), 8), NJ * HB)], gix)

    lanes = lax.iota(jnp.int32, L)
    zeros = jnp.zeros((L,), jnp.float32)
    gbs = (gb0, gb1)
    ros = (ro0, ro1)
    rps = (rp0, rp1)

    def gather_src(j):
        return ctx_hbm.at[gix.at[pl.ds(pl.multiple_of(j * HB, 8), HB)]]

    pltpu.async_copy(gather_src(0), gb0, gsem)

    pltpu.sync_copy(mask_hbm.at[pl.ds(pl.multiple_of(base, 8), BPW)], msk_v)
    pltpu.sync_copy(dir_hbm, d_v)
    pltpu.sync_copy(pre_hbm, cbuf.at[pl.ds(0, 1), pl.ds(0, PRE), :])
    pltpu.sync_copy(suf_hbm, cbuf.at[pl.ds(0, 1), pl.ds(PRE + 2, SUF), :])

    for q, rp in enumerate(rps):
        cr = wid + NW * q

        @pl.when(cr < NCONST)
        def _():
            u = jnp.where(cr < PRE, cr, cr + 2)
            t = jnp.where(cr < PRE, cr, K + cr)

            @pl.loop(0, REP)
            def _rep_s(s2):
                @pl.loop(0, CL, unroll=8)
                def _rep_c(c2):
                    rp[0, s2, pl.ds(c2 * L, L)] = cbuf[0, u, pl.ds(c2 * L, L)]

            @pl.loop(0, B // REP)
            def _fire(jb):
                pltpu.async_copy(
                    rp,
                    out_hbm.at[pl.ds(t, 1),
                               pl.ds(pl.multiple_of(jb * REP, 8), REP), :],
                    csem)

    _dnums = lax.GatherDimensionNumbers(
        offset_dims=(), collapsed_slice_dims=(0,), start_index_map=(0,))

    def lane_take(x, idx):
        return lax.gather(x, idx[:, None], _dnums, slice_sizes=(1,),
                          mode=lax.GatherScatterMode.PROMISE_IN_BOUNDS)

    mask_lo = msk_v[pl.ds(0, L)]
    mask_hi = msk_v[pl.ds(L, L)]

    rr_count = [0]

    def handle(k, rh, gb, gbn, j_next_ok):
        j = 2 * k + rh
        pltpu.make_async_copy(gather_src(j), gb, gsem).wait()

        @pl.when(j_next_ok)
        def _():
            pltpu.async_copy(gather_src(j + 1), gbn, gsem)

        b0 = pl.multiple_of(base + HB * k, 8)

        for rr in range(HB):
            ro = ros[rr_count[0] % 2]
            rr_count[0] += 1

            def _drain_ro(ro=ro):
                pltpu.make_async_copy(
                    ro, out_hbm.at[pl.ds(PRE, 1), pl.ds(b0, HB), :],
                    osem).wait()

            if rr_count[0] > 2:
                _drain_ro()
            else:
                pl.when(k > 0)(_drain_ro)

            @pl.loop(0, HB)
            def _sample(s2):
                g = HB * k + s2
                mchunk = jnp.where(g < L, mask_lo, mask_hi)
                m_splat = lane_take(mchunk,
                                    jnp.broadcast_to(g & (L - 1), (L,)))

                accs = [zeros, zeros, zeros, zeros]
                for c2 in range(CL):
                    accs[c2 % 4] = accs[c2 % 4] + (
                        gb[s2, rr, pl.ds(c2 * L, L)]
                        * d_v[0, pl.ds(c2 * L, L)])
                acc = (accs[0] + accs[1]) + (accs[2] + accs[3])
                for h in (8, 4, 2, 1):
                    acc = acc + lane_take(acc, lanes ^ h)
                sv = m_splat * acc

                for c2 in range(CL):
                    ro[0, s2, pl.ds(c2 * L, L)] = (
                        gb[s2, rr, pl.ds(c2 * L, L)]
                        - sv * d_v[0, pl.ds(c2 * L, L)])

            pltpu.async_copy(
                ro, out_hbm.at[pl.ds(PRE + 8 * rh + rr, 1),
                               pl.ds(b0, HB), :], osem)

    @pl.loop(0, BPW // HB)
    def _block(k):
        handle(k, 0, gb0, gb1, 2 * k + 1 < NJ)
        handle(k, 1, gb1, gb0, 2 * k + 2 < NJ)

    def out_drain():
        pltpu.make_async_copy(
            ro0, out_hbm.at[pl.ds(PRE, 1),
                            pl.ds(pl.multiple_of(base, 8), HB), :],
            osem).wait()

    out_drain()
    out_drain()

    for q, rp in enumerate(rps):
        cr = wid + NW * q

        @pl.when(cr < NCONST)
        def _():
            @pl.loop(0, B // REP)
            def _drain(jb):
                pltpu.make_async_copy(
                    rp, out_hbm.at[pl.ds(0, 1), pl.ds(0, REP), :],
                    csem).wait()


def kernel(labels, cls_ctx, token_prefix, token_suffix, cloth_direction):
    labels_i = labels.astype(jnp.int32)
    lab_blocks = labels_i.reshape(NW, BPW // HB, 1, HB)
    halves = jnp.arange(2, dtype=jnp.int32).reshape(1, 1, 2, 1)
    gidx = (lab_blocks * 2 + halves).reshape(NW * NJ * HB)
    ctx_rows = cls_ctx.reshape(NUM_CLASS * 2, K // 2, C)
    mask = (jax.random.uniform(jax.random.key(42), (B,)) < MASK_PROB)
    mask = mask.astype(jnp.float32)
    out_t = _sc_prompts(gidx, ctx_rows, token_prefix, token_suffix,
                        cloth_direction, mask)
    return jnp.transpose(out_t, (1, 0, 2))


# speedup vs baseline: 1.2090x; 1.2090x over previous
import functools

import jax
import jax.numpy as jnp
from jax import lax
from jax.experimental import pallas as pl
from jax.experimental.pallas import tpu as pltpu
from jax.experimental.pallas import tpu_sc as plsc

B = 1024
NUM_CLASS = 1000
K = 16
C = 512
PRE = 6
SUF = 55
T = PRE + K + SUF
NCONST = PRE + SUF
MASK_PROB = 0.5

NC, NS, L = 2, 16, 16
NW = NC * NS
BPW = B // NW
CL = C // L
REP = 16

_mesh = plsc.VectorSubcoreMesh(
    core_axis_name="c", subcore_axis_name="s", num_cores=NC, num_subcores=NS
)


@functools.partial(
    pl.kernel,
    out_type=jax.ShapeDtypeStruct((T, B, C), jnp.float32),
    mesh=_mesh,
    scratch_types=[
        pltpu.VMEM((BPW,), jnp.int32),
        pltpu.VMEM((BPW,), jnp.int32),
        pltpu.VMEM((BPW,), jnp.int32),
        pltpu.VMEM((BPW,), jnp.float32),
        pltpu.VMEM((1, C), jnp.float32),
        pltpu.VMEM((BPW, C), jnp.float32),
        pltpu.VMEM((BPW, C), jnp.float32),
        pltpu.VMEM((1, BPW, C), jnp.float32),
        pltpu.VMEM((1, BPW, C), jnp.float32),
        pltpu.VMEM((1, PRE + 2 + SUF, C), jnp.float32),
        pltpu.VMEM((1, REP, C), jnp.float32),
        pltpu.VMEM((1, REP, C), jnp.float32),
        pltpu.SemaphoreType.DMA,
        pltpu.SemaphoreType.DMA,
        pltpu.SemaphoreType.DMA,
    ],
)
def _sc_prompts(labm_hbm, ctx_hbm, pre_hbm, suf_hbm, dir_hbm, mask_hbm,
                out_hbm, lab_v, ix0, ix1, msk_v, d_v, gr0, gr1, ro0, ro1,
                cbuf, rp0, rp1, gsem, osem, csem):
    wid = lax.axis_index("s") * NC + lax.axis_index("c")
    base = wid * BPW

    pltpu.sync_copy(labm_hbm.at[pl.ds(pl.multiple_of(base, 8), BPW)], lab_v)

    lanes = lax.iota(jnp.int32, L)
    zeros = jnp.zeros((L,), jnp.float32)

    ixs = (ix0, ix1)
    grs = (gr0, gr1)
    ros = (ro0, ro1)
    rps = (rp0, rp1)

    def set_row_indices(r, ix):
        ix[pl.ds(0, L)] = lab_v[pl.ds(0, L)] + r
        ix[pl.ds(L, L)] = lab_v[pl.ds(L, L)] + r

    set_row_indices(0, ix0)
    pltpu.async_copy(ctx_hbm.at[ix0], gr0, gsem)

    pltpu.sync_copy(mask_hbm.at[pl.ds(pl.multiple_of(base, 8), BPW)], msk_v)
    pltpu.sync_copy(dir_hbm, d_v)
    pltpu.sync_copy(pre_hbm, cbuf.at[pl.ds(0, 1), pl.ds(0, PRE), :])
    pltpu.sync_copy(suf_hbm, cbuf.at[pl.ds(0, 1), pl.ds(PRE + 2, SUF), :])

    for q, rp in enumerate(rps):
        cr = wid + NW * q

        @pl.when(cr < NCONST)
        def _():
            u = jnp.where(cr < PRE, cr, cr + 2)
            t = jnp.where(cr < PRE, cr, K + cr)

            @pl.loop(0, REP)
            def _rep_s(s2):
                @pl.loop(0, CL, unroll=8)
                def _rep_c(c2):
                    rp[0, s2, pl.ds(c2 * L, L)] = cbuf[0, u, pl.ds(c2 * L, L)]

    _dnums = lax.GatherDimensionNumbers(
        offset_dims=(), collapsed_slice_dims=(0,), start_index_map=(0,))

    def lane_take(x, idx):
        return lax.gather(x, idx[:, None], _dnums, slice_sizes=(1,),
                          mode=lax.GatherScatterMode.PROMISE_IN_BOUNDS)

    mask_lo = msk_v[pl.ds(0, L)]
    mask_hi = msk_v[pl.ds(L, L)]

    def handle(r, ix, ixn, gr, grn, ro, has_next, drain_out):
        pltpu.make_async_copy(ctx_hbm.at[ix], gr, gsem).wait()

        @pl.when(has_next)
        def _():
            pltpu.async_copy(ctx_hbm.at[ixn], grn, gsem)

        @pl.when(drain_out)
        def _():
            pltpu.make_async_copy(
                ro, out_hbm.at[pl.ds(PRE, 1),
                               pl.ds(pl.multiple_of(base, 8), BPW), :],
                osem).wait()

        @pl.loop(0, BPW)
        def _sample(s):
            mchunk = jnp.where(s < L, mask_lo, mask_hi)
            m_splat = lane_take(mchunk, jnp.broadcast_to(s & (L - 1), (L,)))

            def dot_body(c2, acc):
                return acc + gr[s, pl.ds(c2 * L, L)] * d_v[0, pl.ds(c2 * L, L)]
            acc = lax.fori_loop(0, CL, dot_body, zeros, unroll=8)
            for h in (8, 4, 2, 1):
                acc = acc + lane_take(acc, lanes ^ h)
            sv = m_splat * acc

            @pl.loop(0, CL, unroll=8)
            def _upd(c2):
                ro[0, s, pl.ds(c2 * L, L)] = (
                    gr[s, pl.ds(c2 * L, L)] - sv * d_v[0, pl.ds(c2 * L, L)])

        pltpu.async_copy(
            ro, out_hbm.at[pl.ds(PRE + r, 1),
                           pl.ds(pl.multiple_of(base, 8), BPW), :], osem)

        for q, rp in enumerate(rps):
            cr = wid + NW * q
            t = jnp.where(cr < PRE, cr, K + cr)

            @pl.when(cr < NCONST)
            def _():
                for f in range(4):
                    jb = 4 * r + f

                    pltpu.async_copy(
                        rp,
                        out_hbm.at[pl.ds(t, 1),
                                   pl.ds(pl.multiple_of(jb * REP, 8),
                                         REP), :],
                        csem)

    @pl.loop(0, K // 2)
    def _pair(g):
        a = 2 * g

        @pl.when(a + 1 < K)
        def _():
            set_row_indices(a + 1, ix1)
        handle(a, ix0, ix1, gr0, gr1, ro0, a + 1 < K, a >= 2)

        @pl.when(a + 2 < K)
        def _():
            set_row_indices(a + 2, ix0)
        handle(a + 1, ix1, ix0, gr1, gr0, ro1, a + 2 < K, a >= 2)

    def out_drain():
        pltpu.make_async_copy(
            ro0, out_hbm.at[pl.ds(PRE, 1),
                            pl.ds(pl.multiple_of(base, 8), BPW), :],
            osem).wait()

    out_drain()
    out_drain()

    for q, rp in enumerate(rps):
        cr = wid + NW * q

        @pl.when(cr < NCONST)
        def _():
            @pl.loop(0, B // REP)
            def _drain(jb):
                pltpu.make_async_copy(
                    rp, out_hbm.at[pl.ds(0, 1), pl.ds(0, REP), :],
                    csem).wait()


def kernel(labels, cls_ctx, token_prefix, token_suffix, cloth_direction):
    labm = labels.astype(jnp.int32) * K
    ctx_rows = cls_ctx.reshape(NUM_CLASS * K, C)
    mask = (jax.random.uniform(jax.random.key(42), (B,)) < MASK_PROB)
    mask = mask.astype(jnp.float32)
    out_t = _sc_prompts(labm, ctx_rows, token_prefix, token_suffix,
                        cloth_direction, mask)
    return jnp.transpose(out_t, (1, 0, 2))


# speedup vs baseline: 1.2210x; 1.0100x over previous
import functools

import jax
import jax.numpy as jnp
from jax import lax
from jax.experimental import pallas as pl
from jax.experimental.pallas import tpu as pltpu
from jax.experimental.pallas import tpu_sc as plsc

B = 1024
NUM_CLASS = 1000
K = 16
C = 512
PRE = 6
SUF = 55
T = PRE + K + SUF
NCONST = PRE + SUF
MASK_PROB = 0.5

NC, NS, L = 2, 16, 16
NW = NC * NS
BPW = B // NW
CL = C // L
REP = 16

_mesh = plsc.VectorSubcoreMesh(
    core_axis_name="c", subcore_axis_name="s", num_cores=NC, num_subcores=NS
)


@functools.partial(
    pl.kernel,
    out_type=jax.ShapeDtypeStruct((T, B, C), jnp.float32),
    mesh=_mesh,
    scratch_types=[
        pltpu.VMEM((BPW,), jnp.int32),
        pltpu.VMEM((BPW,), jnp.int32),
        pltpu.VMEM((BPW,), jnp.int32),
        pltpu.VMEM((BPW,), jnp.float32),
        pltpu.VMEM((1, C), jnp.float32),
        pltpu.VMEM((BPW, C), jnp.float32),
        pltpu.VMEM((BPW, C), jnp.float32),
        pltpu.VMEM((1, BPW, C), jnp.float32),
        pltpu.VMEM((1, BPW, C), jnp.float32),
        pltpu.VMEM((1, PRE + 2 + SUF, C), jnp.float32),
        pltpu.VMEM((1, REP, C), jnp.float32),
        pltpu.VMEM((1, REP, C), jnp.float32),
        pltpu.SemaphoreType.DMA,
        pltpu.SemaphoreType.DMA,
        pltpu.SemaphoreType.DMA,
    ],
)
def _sc_prompts(labm_hbm, ctx_hbm, pre_hbm, suf_hbm, dir_hbm, mask_hbm,
                out_hbm, lab_v, ix0, ix1, msk_v, d_v, gr0, gr1, ro0, ro1,
                cbuf, rp0, rp1, gsem, osem, csem):
    wid = lax.axis_index("s") * NC + lax.axis_index("c")
    base = wid * BPW

    pltpu.sync_copy(labm_hbm.at[pl.ds(pl.multiple_of(base, 8), BPW)], lab_v)

    lanes = lax.iota(jnp.int32, L)
    zeros = jnp.zeros((L,), jnp.float32)

    ixs = (ix0, ix1)
    grs = (gr0, gr1)
    ros = (ro0, ro1)
    rps = (rp0, rp1)

    def set_row_indices(r, ix):
        ix[pl.ds(0, L)] = lab_v[pl.ds(0, L)] + r
        ix[pl.ds(L, L)] = lab_v[pl.ds(L, L)] + r

    set_row_indices(0, ix0)
    pltpu.async_copy(ctx_hbm.at[ix0], gr0, gsem)

    pltpu.sync_copy(mask_hbm.at[pl.ds(pl.multiple_of(base, 8), BPW)], msk_v)
    pltpu.sync_copy(dir_hbm, d_v)
    pltpu.sync_copy(pre_hbm, cbuf.at[pl.ds(0, 1), pl.ds(0, PRE), :])
    pltpu.sync_copy(suf_hbm, cbuf.at[pl.ds(0, 1), pl.ds(PRE + 2, SUF), :])

    for q, rp in enumerate(rps):
        cr = wid + NW * q

        @pl.when(cr < NCONST)
        def _():
            u = jnp.where(cr < PRE, cr, cr + 2)
            t = jnp.where(cr < PRE, cr, K + cr)

            @pl.loop(0, REP)
            def _rep_s(s2):
                @pl.loop(0, CL, unroll=8)
                def _rep_c(c2):
                    rp[0, s2, pl.ds(c2 * L, L)] = cbuf[0, u, pl.ds(c2 * L, L)]

    _dnums = lax.GatherDimensionNumbers(
        offset_dims=(), collapsed_slice_dims=(0,), start_index_map=(0,))

    def lane_take(x, idx):
        return lax.gather(x, idx[:, None], _dnums, slice_sizes=(1,),
                          mode=lax.GatherScatterMode.PROMISE_IN_BOUNDS)

    mask_lo = msk_v[pl.ds(0, L)]
    mask_hi = msk_v[pl.ds(L, L)]

    def handle(r, ix, ixn, gr, grn, ro, has_next, drain_out):
        pltpu.make_async_copy(ctx_hbm.at[ix], gr, gsem).wait()

        @pl.when(has_next)
        def _():
            pltpu.async_copy(ctx_hbm.at[ixn], grn, gsem)

        @pl.when(drain_out)
        def _():
            pltpu.make_async_copy(
                ro, out_hbm.at[pl.ds(PRE, 1),
                               pl.ds(pl.multiple_of(base, 8), BPW), :],
                osem).wait()

        @pl.loop(0, BPW)
        def _sample(s):
            mchunk = jnp.where(s < L, mask_lo, mask_hi)
            m_splat = lane_take(mchunk, jnp.broadcast_to(s & (L - 1), (L,)))

            accs = [zeros, zeros, zeros, zeros]
            for c2 in range(CL):
                accs[c2 % 4] = accs[c2 % 4] + (
                    gr[s, pl.ds(c2 * L, L)] * d_v[0, pl.ds(c2 * L, L)])
            acc = (accs[0] + accs[1]) + (accs[2] + accs[3])
            for h in (8, 4, 2, 1):
                acc = acc + lane_take(acc, lanes ^ h)
            sv = m_splat * acc

            for c2 in range(CL):
                ro[0, s, pl.ds(c2 * L, L)] = (
                    gr[s, pl.ds(c2 * L, L)] - sv * d_v[0, pl.ds(c2 * L, L)])

        pltpu.async_copy(
            ro, out_hbm.at[pl.ds(PRE + r, 1),
                           pl.ds(pl.multiple_of(base, 8), BPW), :], osem)

        for q, rp in enumerate(rps):
            cr = wid + NW * q
            t = jnp.where(cr < PRE, cr, K + cr)

            @pl.when(cr < NCONST)
            def _():
                for f in range(4):
                    jb = 4 * r + f

                    pltpu.async_copy(
                        rp,
                        out_hbm.at[pl.ds(t, 1),
                                   pl.ds(pl.multiple_of(jb * REP, 8),
                                         REP), :],
                        csem)

    @pl.loop(0, K // 2)
    def _pair(g):
        a = 2 * g

        @pl.when(a + 1 < K)
        def _():
            set_row_indices(a + 1, ix1)
        handle(a, ix0, ix1, gr0, gr1, ro0, a + 1 < K, a >= 2)

        @pl.when(a + 2 < K)
        def _():
            set_row_indices(a + 2, ix0)
        handle(a + 1, ix1, ix0, gr1, gr0, ro1, a + 2 < K, a >= 2)

    def out_drain():
        pltpu.make_async_copy(
            ro0, out_hbm.at[pl.ds(PRE, 1),
                            pl.ds(pl.multiple_of(base, 8), BPW), :],
            osem).wait()

    out_drain()
    out_drain()

    for q, rp in enumerate(rps):
        cr = wid + NW * q

        @pl.when(cr < NCONST)
        def _():
            @pl.loop(0, B // REP)
            def _drain(jb):
                pltpu.make_async_copy(
                    rp, out_hbm.at[pl.ds(0, 1), pl.ds(0, REP), :],
                    csem).wait()


def kernel(labels, cls_ctx, token_prefix, token_suffix, cloth_direction):
    labm = labels.astype(jnp.int32) * K
    ctx_rows = cls_ctx.reshape(NUM_CLASS * K, C)
    mask = (jax.random.uniform(jax.random.key(42), (B,)) < MASK_PROB)
    mask = mask.astype(jnp.float32)
    out_t = _sc_prompts(labm, ctx_rows, token_prefix, token_suffix,
                        cloth_direction, mask)
    return jnp.transpose(out_t, (1, 0, 2))
